# baseline (speedup 1.0000x reference)
import functools

import jax
import jax.numpy as jnp
from jax import lax
from jax.experimental import pallas as pl
from jax.experimental.pallas import tpu as pltpu
from jax.experimental.pallas import tpu_sc as plsc

_MESH = plsc.ScalarSubcoreMesh(axis_name="c", num_cores=1)

_PERM = [(i + 4 * (i % 4)) % 16 for i in range(16)]


def _xt(x):
    return ((x << 1) ^ ((x >> 7) & 1) * 27) & 255


@functools.partial(
    pl.kernel,
    mesh=_MESH,
    compiler_params=pltpu.CompilerParams(
        needs_layout_passes=False,
        disable_bounds_checks=True,
    ),
    out_type=jax.ShapeDtypeStruct((16,), jnp.int32),
    scratch_types=[
        pltpu.SMEM((256,), jnp.int32),
        pltpu.SMEM((10,), jnp.int32),
        pltpu.SMEM((16,), jnp.int32),
        pltpu.SMEM((16,), jnp.int32),
        pltpu.SMEM((16,), jnp.int32),
    ],
)
def _aes_scs(pt_hbm, key_hbm, sbox_hbm, rcon_hbm, out_hbm,
             sbox_s, rcon_s, st_s, rk_s, tmp_s):
    pltpu.sync_copy(sbox_hbm, sbox_s)
    pltpu.sync_copy(rcon_hbm, rcon_s)
    pltpu.sync_copy(pt_hbm, st_s)
    pltpu.sync_copy(key_hbm, rk_s)

    for i in range(16):
        st_s[i] = st_s[i] ^ rk_s[i]

    def _next_round_key(r):
        t0 = sbox_s[rk_s[13]] ^ rcon_s[r - 1]
        t1 = sbox_s[rk_s[14]]
        t2 = sbox_s[rk_s[15]]
        t3 = sbox_s[rk_s[12]]
        rk_s[0] = rk_s[0] ^ t0
        rk_s[1] = rk_s[1] ^ t1
        rk_s[2] = rk_s[2] ^ t2
        rk_s[3] = rk_s[3] ^ t3
        for i in range(4, 16):
            rk_s[i] = rk_s[i] ^ rk_s[i - 4]

    def _round(r, _):
        _next_round_key(r)
        for i in range(16):
            tmp_s[i] = sbox_s[st_s[_PERM[i]]]
        for c in range(4):
            a0 = tmp_s[4 * c]
            a1 = tmp_s[4 * c + 1]
            a2 = tmp_s[4 * c + 2]
            a3 = tmp_s[4 * c + 3]
            t = a0 ^ a1 ^ a2 ^ a3
            st_s[4 * c] = a0 ^ t ^ _xt(a0 ^ a1) ^ rk_s[4 * c]
            st_s[4 * c + 1] = a1 ^ t ^ _xt(a1 ^ a2) ^ rk_s[4 * c + 1]
            st_s[4 * c + 2] = a2 ^ t ^ _xt(a2 ^ a3) ^ rk_s[4 * c + 2]
            st_s[4 * c + 3] = a3 ^ t ^ _xt(a3 ^ a0) ^ rk_s[4 * c + 3]
        return 0

    lax.fori_loop(1, 10, _round, 0, unroll=False)

    _next_round_key(10)
    for i in range(16):
        tmp_s[i] = sbox_s[st_s[_PERM[i]]] ^ rk_s[i]
    pltpu.sync_copy(tmp_s, out_hbm)


def kernel(plaintext, key, sbox, rcon):
    return _aes_scs(plaintext, key, sbox, rcon)


# speedup vs baseline: 1.0805x; 1.0805x over previous
import functools

import jax
import jax.numpy as jnp
from jax import lax
from jax.experimental import pallas as pl
from jax.experimental.pallas import tpu as pltpu
from jax.experimental.pallas import tpu_sc as plsc

_MESH = plsc.ScalarSubcoreMesh(axis_name="c", num_cores=1)

_PERM = [(i + 4 * (i % 4)) % 16 for i in range(16)]


def _xt(x):
    return ((x << 1) ^ ((x >> 7) & 1) * 27) & 255


@functools.partial(
    pl.kernel,
    mesh=_MESH,
    compiler_params=pltpu.CompilerParams(
        needs_layout_passes=False,
        disable_bounds_checks=True,
    ),
    out_type=jax.ShapeDtypeStruct((16,), jnp.int32),
    scratch_types=[
        pltpu.SMEM((256,), jnp.int32),
        pltpu.SMEM((10,), jnp.int32),
        pltpu.SMEM((16,), jnp.int32),
        pltpu.SMEM((16,), jnp.int32),
    ],
)
def _aes_scs(pt_hbm, key_hbm, sbox_hbm, rcon_hbm, out_hbm,
             sbox_s, rcon_s, io_s, key_s):
    pltpu.sync_copy(sbox_hbm, sbox_s)
    pltpu.sync_copy(rcon_hbm, rcon_s)
    pltpu.sync_copy(pt_hbm, io_s)
    pltpu.sync_copy(key_hbm, key_s)

    rk = [key_s[i] for i in range(16)]
    st = [io_s[i] ^ rk[i] for i in range(16)]

    def _next_round_key(rk, rc):
        t = [sbox_s[rk[13]] ^ rc, sbox_s[rk[14]], sbox_s[rk[15]], sbox_s[rk[12]]]
        out = list(rk)
        for i in range(4):
            out[i] = rk[i] ^ t[i]
        for i in range(4, 16):
            out[i] = out[i - 4] ^ rk[i]
        return out

    def _round(r, carry):
        st = list(carry[:16])
        rk = _next_round_key(list(carry[16:]), rcon_s[r - 1])
        sb = [sbox_s[st[_PERM[i]]] for i in range(16)]
        for c in range(4):
            a0, a1, a2, a3 = sb[4 * c:4 * c + 4]
            t = a0 ^ a1 ^ a2 ^ a3
            st[4 * c] = a0 ^ t ^ _xt(a0 ^ a1) ^ rk[4 * c]
            st[4 * c + 1] = a1 ^ t ^ _xt(a1 ^ a2) ^ rk[4 * c + 1]
            st[4 * c + 2] = a2 ^ t ^ _xt(a2 ^ a3) ^ rk[4 * c + 2]
            st[4 * c + 3] = a3 ^ t ^ _xt(a3 ^ a0) ^ rk[4 * c + 3]
        return tuple(st) + tuple(rk)

    carry = lax.fori_loop(1, 10, _round, tuple(st) + tuple(rk), unroll=False)
    st = list(carry[:16])
    rk = _next_round_key(list(carry[16:]), rcon_s[9])
    for i in range(16):
        io_s[i] = sbox_s[st[_PERM[i]]] ^ rk[i]
    pltpu.sync_copy(io_s, out_hbm)


def kernel(plaintext, key, sbox, rcon):
    return _aes_scs(plaintext, key, sbox, rcon)


# speedup vs baseline: 1.1751x; 1.0875x over previous
import functools

import jax
import jax.numpy as jnp
from jax import lax
from jax.experimental import pallas as pl
from jax.experimental.pallas import tpu as pltpu
from jax.experimental.pallas import tpu_sc as plsc

_MESH = plsc.ScalarSubcoreMesh(axis_name="c", num_cores=1)

_PERM = [(i + 4 * (i % 4)) % 16 for i in range(16)]


def _xt(x):
    return ((x << 1) ^ ((x >> 7) & 1) * 27) & 255


@functools.partial(
    pl.kernel,
    mesh=_MESH,
    compiler_params=pltpu.CompilerParams(
        needs_layout_passes=False,
        disable_bounds_checks=True,
    ),
    out_type=jax.ShapeDtypeStruct((16,), jnp.int32),
    scratch_types=[
        pltpu.SMEM((256,), jnp.int32),
        pltpu.SMEM((10,), jnp.int32),
        pltpu.SMEM((16,), jnp.int32),
        pltpu.SMEM((16,), jnp.int32),
        pltpu.SemaphoreType.DMA,
    ],
)
def _aes_scs(pt_hbm, key_hbm, sbox_hbm, rcon_hbm, out_hbm,
             sbox_s, rcon_s, io_s, key_s, sem):
    c1 = pltpu.async_copy(sbox_hbm, sbox_s, sem)
    c2 = pltpu.async_copy(rcon_hbm, rcon_s, sem)
    c3 = pltpu.async_copy(pt_hbm, io_s, sem)
    c4 = pltpu.async_copy(key_hbm, key_s, sem)
    c1.wait()
    c2.wait()
    c3.wait()
    c4.wait()

    rk = [key_s[i] for i in range(16)]
    st = [io_s[i] ^ rk[i] for i in range(16)]

    def _next_round_key(rk, rc):
        t = [sbox_s[rk[13]] ^ rc, sbox_s[rk[14]], sbox_s[rk[15]], sbox_s[rk[12]]]
        out = list(rk)
        for i in range(4):
            out[i] = rk[i] ^ t[i]
        for i in range(4, 16):
            out[i] = out[i - 4] ^ rk[i]
        return out

    def _round(r, carry):
        st = list(carry[:16])
        rk = _next_round_key(list(carry[16:]), rcon_s[r - 1])
        sb = [sbox_s[st[_PERM[i]]] for i in range(16)]
        for c in range(4):
            a0, a1, a2, a3 = sb[4 * c:4 * c + 4]
            t = a0 ^ a1 ^ a2 ^ a3
            st[4 * c] = a0 ^ t ^ _xt(a0 ^ a1) ^ rk[4 * c]
            st[4 * c + 1] = a1 ^ t ^ _xt(a1 ^ a2) ^ rk[4 * c + 1]
            st[4 * c + 2] = a2 ^ t ^ _xt(a2 ^ a3) ^ rk[4 * c + 2]
            st[4 * c + 3] = a3 ^ t ^ _xt(a3 ^ a0) ^ rk[4 * c + 3]
        return tuple(st) + tuple(rk)

    carry = lax.fori_loop(1, 10, _round, tuple(st) + tuple(rk), unroll=False)
    st = list(carry[:16])
    rk = _next_round_key(list(carry[16:]), rcon_s[9])
    for i in range(16):
        io_s[i] = sbox_s[st[_PERM[i]]] ^ rk[i]
    pltpu.sync_copy(io_s, out_hbm)


def kernel(plaintext, key, sbox, rcon):
    return _aes_scs(plaintext, key, sbox, rcon)


# speedup vs baseline: 1.1774x; 1.0020x over previous
import functools

import jax
import jax.numpy as jnp
from jax import lax
from jax.experimental import pallas as pl
from jax.experimental.pallas import tpu as pltpu
from jax.experimental.pallas import tpu_sc as plsc

_MESH = plsc.ScalarSubcoreMesh(axis_name="c", num_cores=1)

_PERM = [(i + 4 * (i % 4)) % 16 for i in range(16)]


def _xt(x):
    return ((x << 1) ^ ((x >> 7) & 1) * 27) & 255


@functools.partial(
    pl.kernel,
    mesh=_MESH,
    compiler_params=pltpu.CompilerParams(
        needs_layout_passes=False,
        disable_bounds_checks=True,
    ),
    out_type=jax.ShapeDtypeStruct((16,), jnp.int32),
    scratch_types=[
        pltpu.SMEM((256,), jnp.int32),
        pltpu.SMEM((10,), jnp.int32),
        pltpu.SMEM((16,), jnp.int32),
        pltpu.SMEM((16,), jnp.int32),
        pltpu.SemaphoreType.DMA,
    ],
)
def _aes_scs(pt_hbm, key_hbm, sbox_hbm, rcon_hbm, out_hbm,
             sbox_s, rcon_s, io_s, key_s, sem):
    c1 = pltpu.async_copy(sbox_hbm, sbox_s, sem)
    c2 = pltpu.async_copy(rcon_hbm, rcon_s, sem)
    c3 = pltpu.async_copy(pt_hbm, io_s, sem)
    c4 = pltpu.async_copy(key_hbm, key_s, sem)
    c1.wait()
    c2.wait()
    c3.wait()
    c4.wait()

    rk = [key_s[i] for i in range(16)]
    st = [io_s[i] ^ rk[i] for i in range(16)]

    def _next_round_key(rk, rc):
        t = [sbox_s[rk[13]] ^ rc, sbox_s[rk[14]], sbox_s[rk[15]], sbox_s[rk[12]]]
        out = list(rk)
        for i in range(4):
            out[i] = rk[i] ^ t[i]
        for i in range(4, 16):
            out[i] = out[i - 4] ^ rk[i]
        return out

    def _round(r, carry):
        st = list(carry[:16])
        rk = _next_round_key(list(carry[16:]), rcon_s[r - 1])
        sb = [sbox_s[st[_PERM[i]]] for i in range(16)]
        for c in range(4):
            a0, a1, a2, a3 = sb[4 * c:4 * c + 4]
            t = a0 ^ a1 ^ a2 ^ a3
            st[4 * c] = a0 ^ t ^ _xt(a0 ^ a1) ^ rk[4 * c]
            st[4 * c + 1] = a1 ^ t ^ _xt(a1 ^ a2) ^ rk[4 * c + 1]
            st[4 * c + 2] = a2 ^ t ^ _xt(a2 ^ a3) ^ rk[4 * c + 2]
            st[4 * c + 3] = a3 ^ t ^ _xt(a3 ^ a0) ^ rk[4 * c + 3]
        return tuple(st) + tuple(rk)

    carry = tuple(st) + tuple(rk)
    for r in range(1, 10):
        carry = _round(r, carry)
    st = list(carry[:16])
    rk = _next_round_key(list(carry[16:]), rcon_s[9])
    for i in range(16):
        io_s[i] = sbox_s[st[_PERM[i]]] ^ rk[i]
    pltpu.sync_copy(io_s, out_hbm)


def kernel(plaintext, key, sbox, rcon):
    return _aes_scs(plaintext, key, sbox, rcon)
